# baseline (speedup 1.0000x reference)
import functools

import jax
import jax.numpy as jnp
from jax import lax
from jax.experimental import pallas as pl
from jax.experimental.pallas import tpu as pltpu
from jax.experimental.pallas import tpu_sc as plsc

D = 64
G = 16384
NC = 2
NS = 16
NW = NC * NS
BPW = G // NW
NCH = 4
CH = BPW // NCH
L = 16


def _build_sc_kernel():
    mesh = plsc.VectorSubcoreMesh(core_axis_name="c", subcore_axis_name="s")

    @functools.partial(
        pl.kernel,
        mesh=mesh,
        compiler_params=pltpu.CompilerParams(use_tc_tiling_on_sc=False),
        out_type=jax.ShapeDtypeStruct((NW, BPW, D), jnp.float32),
        scratch_types=[
            pltpu.VMEM((NCH, CH), jnp.int32),
            pltpu.VMEM((NCH, CH), jnp.int32),
            pltpu.VMEM((BPW,), jnp.float32),
            pltpu.VMEM((BPW, D), jnp.float32),
            pltpu.VMEM((BPW, D), jnp.float32),
            pltpu.SemaphoreType.DMA,
            pltpu.SemaphoreType.DMA,
        ],
    )
    def k(idx_hbm, midx_hbm, mask_hbm, pe_hbm, mt_hbm, out_hbm,
          idx_v, midx_v, mask_v, rows_pe, rows_m, semp, semm):
        wid = lax.axis_index("s") * NC + lax.axis_index("c")

        pltpu.sync_copy(idx_hbm.at[wid], idx_v)
        pltpu.sync_copy(midx_hbm.at[wid], midx_v)
        pltpu.sync_copy(mask_hbm.at[wid], mask_v)

        copies = []
        for c in range(NCH):
            copies.append(
                pltpu.async_copy(pe_hbm.at[idx_v.at[c]],
                                 rows_pe.at[pl.ds(c * CH, CH)], semp))
            copies.append(
                pltpu.async_copy(mt_hbm.at[midx_v.at[c]],
                                 rows_m.at[pl.ds(c * CH, CH)], semm))
        for cp in copies:
            cp.wait()

        def grp(g, carry):
            mvec = mask_v[pl.ds(g * L, L)]
            for k in range(L):
                gl = g * L + k
                m = mvec[k]
                for j in range(D // L):
                    sl = pl.ds(L * j, L)
                    pe = rows_pe[gl, sl]
                    ms = rows_m[gl, sl]
                    rows_pe[gl, sl] = m * (pe - ms) + ms
            return carry

        lax.fori_loop(0, BPW // L, grp, 0)

        pltpu.sync_copy(rows_pe, out_hbm.at[wid])

    return k


@jax.jit
def kernel(indices, present_mask, missing_idx_map, pe_table, missing_table):
    idx = indices.astype(jnp.int32).reshape(NW, NCH, CH)
    midx = missing_idx_map.astype(jnp.int32).reshape(NW, NCH, CH)
    mask = present_mask.astype(jnp.float32).reshape(NW, BPW)
    out = _build_sc_kernel()(idx, midx, mask, pe_table, missing_table)
    return out.reshape(G, D)
